# baseline (speedup 1.0000x reference)
import jax
import jax.numpy as jnp
from jax.experimental import pallas as pl

B, T, V, D = 4096, 8, 1024, 256
BB = 128


def _decode_block(msg_ref, cb_ref, out_ref):
    m = msg_ref[...]
    mx = jnp.max(m, axis=-1, keepdims=True)
    iota3 = jax.lax.broadcasted_iota(jnp.int32, (BB, T, V), 2)
    codes = jnp.min(jnp.where(m == mx, iota3, V), axis=-1)
    iota2 = jax.lax.broadcasted_iota(jnp.int32, (BB, V), 1)
    acc = jnp.zeros((BB, D), jnp.float32)
    for t in range(T):
        onehot = (iota2 == codes[:, t : t + 1]).astype(jnp.bfloat16)
        g = jax.lax.dot(onehot, cb_ref[t], preferred_element_type=jnp.float32)
        acc = acc + g
        norm = jnp.sqrt(jnp.sum(acc * acc, axis=-1, keepdims=True))
        out_ref[:, t, :] = acc * (1.0 / jnp.maximum(norm, 1e-12))


@jax.jit
def kernel(message, codebooks):
    cb16 = codebooks.astype(jnp.bfloat16)
    return pl.pallas_call(
        _decode_block,
        grid=(B // BB,),
        in_specs=[
            pl.BlockSpec((BB, T, V), lambda i: (i, 0, 0)),
            pl.BlockSpec((T, V, D), lambda i: (0, 0, 0)),
        ],
        out_specs=pl.BlockSpec((BB, T, D), lambda i: (i, 0, 0)),
        out_shape=jax.ShapeDtypeStruct((B, T, D), jnp.float32),
    )(message, cb16)


# speedup vs baseline: 1.1079x; 1.1079x over previous
import jax
import jax.numpy as jnp
from jax.experimental import pallas as pl

B, T, V, D = 4096, 8, 1024, 256
BB = 256


def _decode_block(msg_ref, cb_ref, out_ref):
    m = msg_ref[...]
    mx = jnp.max(m, axis=-1, keepdims=True)
    iota3 = jax.lax.broadcasted_iota(jnp.int32, (BB, T, V), 2)
    codes = jnp.min(jnp.where(m == mx, iota3, V), axis=-1)
    iota2 = jax.lax.broadcasted_iota(jnp.int32, (BB, V), 1)
    acc = jnp.zeros((BB, D), jnp.float32)
    for t in range(T):
        onehot = (iota2 == codes[:, t : t + 1]).astype(jnp.bfloat16)
        g = jax.lax.dot(onehot, cb_ref[t], preferred_element_type=jnp.float32)
        acc = acc + g
        norm = jnp.sqrt(jnp.sum(acc * acc, axis=-1, keepdims=True))
        out_ref[:, t, :] = acc * (1.0 / jnp.maximum(norm, 1e-12))


@jax.jit
def kernel(message, codebooks):
    cb16 = codebooks.astype(jnp.bfloat16)
    return pl.pallas_call(
        _decode_block,
        grid=(B // BB,),
        in_specs=[
            pl.BlockSpec((BB, T, V), lambda i: (i, 0, 0)),
            pl.BlockSpec((T, V, D), lambda i: (0, 0, 0)),
        ],
        out_specs=pl.BlockSpec((BB, T, D), lambda i: (i, 0, 0)),
        out_shape=jax.ShapeDtypeStruct((B, T, D), jnp.float32),
    )(message, cb16)
